# baseline (device time: 17133 ns/iter reference)
import jax
import jax.numpy as jnp
from jax import lax
from jax.experimental import pallas as pl
from jax.experimental.pallas import tpu as pltpu

N_BLOCKS = 4
CHUNK_BLOCKS = 2


def kernel(x, dy, gamma):
    del gamma
    m, d = x.shape
    rows_mine = m // 2
    blk = rows_mine // N_BLOCKS

    def body(
        x_hbm, dy_hbm, out_ref,
        xbuf, dybuf, copy_sems,
        acc_ref, recv_ref, send_sems, recv_sems,
    ):
        my_x = lax.axis_index("x")
        my_y = lax.axis_index("y")
        peers = [
            (my_x, 1 - my_y),
            (1 - my_x, my_y),
            (1 - my_x, 1 - my_y),
        ]

        barrier_sem = pltpu.get_barrier_semaphore()
        for p in peers:
            pl.semaphore_signal(
                barrier_sem, inc=1, device_id=p,
                device_id_type=pl.DeviceIdType.MESH,
            )

        row0 = my_y * rows_mine

        def start_copy(k, slot):
            r = row0 + k * blk
            cx = pltpu.make_async_copy(
                x_hbm.at[pl.ds(r, blk), :], xbuf.at[slot],
                copy_sems.at[slot, 0],
            )
            cdy = pltpu.make_async_copy(
                dy_hbm.at[pl.ds(r, blk), :], dybuf.at[slot],
                copy_sems.at[slot, 1],
            )
            cx.start()
            cdy.start()
            return cx, cdy

        def broadcast(chunk):
            rdmas = []
            for i, p in enumerate(peers):
                r = pltpu.make_async_remote_copy(
                    src_ref=acc_ref.at[chunk],
                    dst_ref=recv_ref.at[chunk, i],
                    send_sem=send_sems.at[chunk, i],
                    recv_sem=recv_sems.at[chunk, i],
                    device_id=p, device_id_type=pl.DeviceIdType.MESH,
                )
                r.start()
                rdmas.append(r)
            return rdmas

        inflight = {0: start_copy(0, 0)}
        rdmas = []
        for k in range(N_BLOCKS):
            slot = k % 2
            chunk = k // CHUNK_BLOCKS
            if k + 1 < N_BLOCKS:
                inflight[k + 1] = start_copy(k + 1, (k + 1) % 2)
            cx, cdy = inflight.pop(k)
            cx.wait()
            cdy.wait()

            xv = xbuf[slot]
            dyv = dybuf[slot]
            mu = jnp.sum(xv, axis=1, keepdims=True) * (1.0 / d)
            xc = xv - mu
            var = jnp.sum(xc * xc, axis=1, keepdims=True) * (1.0 / d)
            xhat = xc * lax.rsqrt(var + 1e-5)
            pg = jnp.sum(dyv * xhat, axis=0)
            pb = jnp.sum(dyv, axis=0)
            if k % CHUNK_BLOCKS == 0:
                acc_ref[chunk, 0, :] = pg
                acc_ref[chunk, 1, :] = pb
            else:
                acc_ref[chunk, 0, :] = acc_ref[chunk, 0, :] + pg
                acc_ref[chunk, 1, :] = acc_ref[chunk, 1, :] + pb

            if k == CHUNK_BLOCKS - 1:
                pl.semaphore_wait(barrier_sem, len(peers))
                rdmas += broadcast(0)

        rdmas += broadcast(1)
        for r in rdmas:
            r.wait()

        own = acc_ref[0, :, :] + acc_ref[1, :, :]
        ra = recv_ref[0, 0] + recv_ref[0, 1] + recv_ref[0, 2]
        rb = recv_ref[1, 0] + recv_ref[1, 1] + recv_ref[1, 2]
        out_ref[:, :] = own + (ra + rb)

    return pl.pallas_call(
        body,
        out_shape=jax.ShapeDtypeStruct((2, d), jnp.float32),
        in_specs=[
            pl.BlockSpec(memory_space=pl.ANY),
            pl.BlockSpec(memory_space=pl.ANY),
        ],
        out_specs=pl.BlockSpec(memory_space=pltpu.VMEM),
        scratch_shapes=[
            pltpu.VMEM((2, blk, d), jnp.float32),
            pltpu.VMEM((2, blk, d), jnp.float32),
            pltpu.SemaphoreType.DMA((2, 2)),
            pltpu.VMEM((2, 2, d), jnp.float32),
            pltpu.VMEM((2, 3, 2, d), jnp.float32),
            pltpu.SemaphoreType.DMA((2, 3)),
            pltpu.SemaphoreType.DMA((2, 3)),
        ],
        compiler_params=pltpu.CompilerParams(collective_id=0),
    )(x, dy)


# device time: 16073 ns/iter; 1.0659x vs baseline; 1.0659x over previous
import jax
import jax.numpy as jnp
from jax import lax
from jax.experimental import pallas as pl
from jax.experimental.pallas import tpu as pltpu

N_BLOCKS = 4


def kernel(x, dy, gamma):
    del gamma
    m, d = x.shape
    rows_mine = m // 2
    blk = rows_mine // N_BLOCKS

    def body(
        x_hbm, dy_hbm, out_ref,
        xbuf, dybuf, copy_sems,
        acc_ref, recv_ref, send_sems, recv_sems,
    ):
        my_x = lax.axis_index("x")
        my_y = lax.axis_index("y")
        peers = [
            (my_x, 1 - my_y),
            (1 - my_x, my_y),
            (1 - my_x, 1 - my_y),
        ]

        barrier_sem = pltpu.get_barrier_semaphore()
        for p in peers:
            pl.semaphore_signal(
                barrier_sem, inc=1, device_id=p,
                device_id_type=pl.DeviceIdType.MESH,
            )

        row0 = my_y * rows_mine

        copies = []
        for k in range(N_BLOCKS):
            r = row0 + k * blk
            cx = pltpu.make_async_copy(
                x_hbm.at[pl.ds(r, blk), :], xbuf.at[k], copy_sems.at[k, 0],
            )
            cdy = pltpu.make_async_copy(
                dy_hbm.at[pl.ds(r, blk), :], dybuf.at[k], copy_sems.at[k, 1],
            )
            cx.start()
            cdy.start()
            copies.append((cx, cdy))

        for k in range(N_BLOCKS):
            cx, cdy = copies[k]
            cx.wait()
            cdy.wait()

            xv = xbuf[k]
            dyv = dybuf[k]
            mu = jnp.sum(xv, axis=1, keepdims=True) * (1.0 / d)
            xc = xv - mu
            var = jnp.sum(xc * xc, axis=1, keepdims=True) * (1.0 / d)
            xhat = xc * lax.rsqrt(var + 1e-5)
            pg = jnp.sum(dyv * xhat, axis=0)
            pb = jnp.sum(dyv, axis=0)
            if k == 0:
                acc_ref[0, :] = pg
                acc_ref[1, :] = pb
            else:
                acc_ref[0, :] = acc_ref[0, :] + pg
                acc_ref[1, :] = acc_ref[1, :] + pb

        pl.semaphore_wait(barrier_sem, len(peers))

        rdmas = []
        for i, p in enumerate(peers):
            r = pltpu.make_async_remote_copy(
                src_ref=acc_ref, dst_ref=recv_ref.at[i],
                send_sem=send_sems.at[i], recv_sem=recv_sems.at[i],
                device_id=p, device_id_type=pl.DeviceIdType.MESH,
            )
            r.start()
            rdmas.append(r)
        for r in rdmas:
            r.wait()

        out_ref[:, :] = (
            (acc_ref[:, :] + recv_ref[0, :, :])
            + (recv_ref[1, :, :] + recv_ref[2, :, :])
        )

    return pl.pallas_call(
        body,
        out_shape=jax.ShapeDtypeStruct((2, d), jnp.float32),
        in_specs=[
            pl.BlockSpec(memory_space=pl.ANY),
            pl.BlockSpec(memory_space=pl.ANY),
        ],
        out_specs=pl.BlockSpec(memory_space=pltpu.VMEM),
        scratch_shapes=[
            pltpu.VMEM((N_BLOCKS, blk, d), jnp.float32),
            pltpu.VMEM((N_BLOCKS, blk, d), jnp.float32),
            pltpu.SemaphoreType.DMA((N_BLOCKS, 2)),
            pltpu.VMEM((2, d), jnp.float32),
            pltpu.VMEM((3, 2, d), jnp.float32),
            pltpu.SemaphoreType.DMA((3,)),
            pltpu.SemaphoreType.DMA((3,)),
        ],
        compiler_params=pltpu.CompilerParams(collective_id=0),
    )(x, dy)
